# baseline (speedup 1.0000x reference)
import functools

import jax
import jax.numpy as jnp
from jax import lax
from jax.experimental import pallas as pl
from jax.experimental.pallas import tpu as pltpu
from jax.experimental.pallas import tpu_sc as plsc

NC = 2
NS = 16
L = 16
NW = NC * NS
H = 768
HB = H // L
TOTAL = 16384
TPW = TOTAL // NW
C = 64
NCH = TPW // C
EPS = 1e-12


def _rsqrt16(v):
    i = plsc.bitcast(v, jnp.int32)
    i = jnp.int32(0x5F3759DF) - lax.shift_right_arithmetic(i, 1)
    y = plsc.bitcast(i, jnp.float32)
    vh = v * jnp.float32(0.5)
    for _ in range(3):
        y = y * (jnp.float32(1.5) - vh * y * y)
    return y


def _body(ids_hbm, pos_hbm, tt_hbm, wtab, ptab, tttab, lnw, lnb, out,
          widx, pidx, tidx, wbuf, pbuf, ttb, lwb, lbb, sem_w, sem_p):
    wid = lax.axis_index("s") * NC + lax.axis_index("c")
    pltpu.sync_copy(tttab, ttb)
    pltpu.sync_copy(lnw, lwb)
    pltpu.sync_copy(lnb, lbb)
    inv_h = jnp.float32(1.0 / H)
    lanes = lax.iota(jnp.int32, L)

    def chunk_body(c, carry):
        base = wid * TPW + c * C
        pltpu.sync_copy(ids_hbm.at[pl.ds(base, C)], widx)
        pltpu.sync_copy(pos_hbm.at[pl.ds(base, C)], pidx)
        pltpu.sync_copy(tt_hbm.at[pl.ds(base, C)], tidx)
        cp_w = pltpu.async_copy(wtab.at[widx], wbuf, sem_w)
        cp_p = pltpu.async_copy(ptab.at[pidx], pbuf, sem_p)
        cp_w.wait()
        cp_p.wait()
        for g in range(C // L):
            tokv = lanes + jnp.int32(g * L)
            ttidv = plsc.load_gather(tidx, [tokv]).astype(jnp.float32)

            def p1(k, kcarry):
                s, q = kcarry
                d0 = k * L
                t0v = ttb[0, pl.ds(d0, L)]
                t1v = ttb[1, pl.ds(d0, L)]
                dtv = t1v - t0v
                for e in range(L):
                    dv = jnp.full((L,), d0 + e, jnp.int32)
                    vw = plsc.load_gather(wbuf, [tokv, dv])
                    vp = plsc.load_gather(pbuf, [tokv, dv])
                    x = vw + vp + (ttidv * dtv[e] + t0v[e])
                    plsc.store_scatter(wbuf, [tokv, dv], x)
                    s = s + x
                    q = q + x * x
                return (s, q)

            zero = jnp.zeros((L,), jnp.float32)
            s, q = lax.fori_loop(0, HB, p1, (zero, zero))
            mean = s * inv_h
            var = q * inv_h - mean * mean
            r = _rsqrt16(var + jnp.float32(EPS))
            nmr = -(mean * r)

            def p2(k, kcarry):
                d0 = k * L
                wv = lwb[pl.ds(d0, L)]
                bv = lbb[pl.ds(d0, L)]
                for e in range(L):
                    dv = jnp.full((L,), d0 + e, jnp.int32)
                    x = plsc.load_gather(wbuf, [tokv, dv])
                    y = x * (r * wv[e]) + (nmr * wv[e] + bv[e])
                    plsc.store_scatter(wbuf, [tokv, dv], y)
                return kcarry

            lax.fori_loop(0, HB, p2, jnp.int32(0))
        pltpu.sync_copy(wbuf, out.at[pl.ds(base, C)])
        return carry

    lax.fori_loop(0, NCH, chunk_body, jnp.int32(0))


@jax.jit
def kernel(input_ids, seq_lens, position_ids, token_type_ids,
           word_embeddings, position_embeddings, token_type_embeddings,
           ln_weight, ln_bias):
    del seq_lens
    mesh = plsc.VectorSubcoreMesh(core_axis_name="c", subcore_axis_name="s")
    kfn = pl.kernel(
        _body,
        out_type=jax.ShapeDtypeStruct((TOTAL, H), jnp.float32),
        mesh=mesh,
        compiler_params=pltpu.CompilerParams(needs_layout_passes=False),
        scratch_types=[
            pltpu.VMEM((C,), jnp.int32),
            pltpu.VMEM((C,), jnp.int32),
            pltpu.VMEM((C,), jnp.int32),
            pltpu.VMEM((C, H), jnp.float32),
            pltpu.VMEM((C, H), jnp.float32),
            pltpu.VMEM((2, H), jnp.float32),
            pltpu.VMEM((H,), jnp.float32),
            pltpu.VMEM((H,), jnp.float32),
            pltpu.SemaphoreType.DMA,
            pltpu.SemaphoreType.DMA,
        ],
    )
    return kfn(input_ids.astype(jnp.int32), position_ids.astype(jnp.int32),
               token_type_ids.astype(jnp.int32), word_embeddings,
               position_embeddings, token_type_embeddings,
               ln_weight, ln_bias)


# speedup vs baseline: 6.3239x; 6.3239x over previous
import functools

import jax
import jax.numpy as jnp
from jax import lax
from jax.experimental import pallas as pl
from jax.experimental.pallas import tpu as pltpu
from jax.experimental.pallas import tpu_sc as plsc

NC = 2
NS = 16
L = 16
NW = NC * NS
H = 768
HB = H // L
TOTAL = 16384
TPW = TOTAL // NW
C = 64
NCH = TPW // C
EPS = 1e-12


def _rsqrt_scalar(v):
    i = lax.bitcast_convert_type(v, jnp.int32)
    i = jnp.int32(0x5F3759DF) - lax.shift_right_arithmetic(i, 1)
    y = lax.bitcast_convert_type(i, jnp.float32)
    vh = v * jnp.float32(0.5)
    for _ in range(3):
        y = y * (jnp.float32(1.5) - vh * y * y)
    return y


def _body(ids_hbm, pos_hbm, tt_hbm, wtab, ptab, tttab, lnw, lnb, out,
          widx, pidx, tidx, wbuf, pbuf, ttb, lwb, lbb, sem_w, sem_p):
    wid = lax.axis_index("s") * NC + lax.axis_index("c")
    pltpu.sync_copy(tttab, ttb)
    pltpu.sync_copy(lnw, lwb)
    pltpu.sync_copy(lnb, lbb)
    inv_h = jnp.float32(1.0 / H)

    def chunk_body(c, carry):
        base = wid * TPW + c * C
        pltpu.sync_copy(ids_hbm.at[pl.ds(base, C)], widx)
        pltpu.sync_copy(pos_hbm.at[pl.ds(base, C)], pidx)
        pltpu.sync_copy(tt_hbm.at[pl.ds(base, C)], tidx)
        cp_w = pltpu.async_copy(wtab.at[widx], wbuf, sem_w)
        cp_p = pltpu.async_copy(ptab.at[pidx], pbuf, sem_p)
        cp_w.wait()
        cp_p.wait()
        for g in range(C // L):
            ttf = tidx[pl.ds(g * L, L)].astype(jnp.float32)
            ttf_s = [ttf[t] for t in range(L)]

            def p1(j, kcarry):
                d0 = j * L
                t0v = ttb[0, pl.ds(d0, L)]
                dtv = ttb[1, pl.ds(d0, L)] - t0v
                new = []
                for t in range(L):
                    tok = g * L + t
                    x = (wbuf[tok, pl.ds(d0, L)] + pbuf[tok, pl.ds(d0, L)]
                         + (dtv * ttf_s[t] + t0v))
                    wbuf[tok, pl.ds(d0, L)] = x
                    new.append(kcarry[2 * t] + x)
                    new.append(kcarry[2 * t + 1] + x * x)
                return tuple(new)

            zero = jnp.zeros((L,), jnp.float32)
            acc = lax.fori_loop(0, HB, p1, (zero,) * (2 * L))
            a_s = []
            c_s = []
            for t in range(L):
                mean = jnp.sum(acc[2 * t]) * inv_h
                var = jnp.sum(acc[2 * t + 1]) * inv_h - mean * mean
                r = _rsqrt_scalar(var + jnp.float32(EPS))
                a_s.append(r)
                c_s.append(-(mean * r))

            def p2(j, kcarry):
                d0 = j * L
                wv = lwb[pl.ds(d0, L)]
                bv = lbb[pl.ds(d0, L)]
                for t in range(L):
                    tok = g * L + t
                    x = wbuf[tok, pl.ds(d0, L)]
                    wbuf[tok, pl.ds(d0, L)] = (x * a_s[t] + c_s[t]) * wv + bv
                return kcarry

            lax.fori_loop(0, HB, p2, jnp.int32(0))
        pltpu.sync_copy(wbuf, out.at[pl.ds(base, C)])
        return carry

    lax.fori_loop(0, NCH, chunk_body, jnp.int32(0))


@jax.jit
def kernel(input_ids, seq_lens, position_ids, token_type_ids,
           word_embeddings, position_embeddings, token_type_embeddings,
           ln_weight, ln_bias):
    del seq_lens
    mesh = plsc.VectorSubcoreMesh(core_axis_name="c", subcore_axis_name="s")
    kfn = pl.kernel(
        _body,
        out_type=jax.ShapeDtypeStruct((TOTAL, H), jnp.float32),
        mesh=mesh,
        compiler_params=pltpu.CompilerParams(needs_layout_passes=False),
        scratch_types=[
            pltpu.VMEM((C,), jnp.int32),
            pltpu.VMEM((C,), jnp.int32),
            pltpu.VMEM((C,), jnp.int32),
            pltpu.VMEM((C, H), jnp.float32),
            pltpu.VMEM((C, H), jnp.float32),
            pltpu.VMEM((2, H), jnp.float32),
            pltpu.VMEM((H,), jnp.float32),
            pltpu.VMEM((H,), jnp.float32),
            pltpu.SemaphoreType.DMA,
            pltpu.SemaphoreType.DMA,
        ],
    )
    return kfn(input_ids.astype(jnp.int32), position_ids.astype(jnp.int32),
               token_type_ids.astype(jnp.int32), word_embeddings,
               position_embeddings, token_type_embeddings,
               ln_weight, ln_bias)


# speedup vs baseline: 7.2441x; 1.1455x over previous
import functools

import jax
import jax.numpy as jnp
from jax import lax
from jax.experimental import pallas as pl
from jax.experimental.pallas import tpu as pltpu
from jax.experimental.pallas import tpu_sc as plsc

NC = 2
NS = 16
L = 16
NW = NC * NS
H = 768
HB = H // L
TOTAL = 16384
TPW = TOTAL // NW
C = 16
NCH = TPW // C
NBUF = 4
EPS = 1e-12


def _rsqrt_scalar(v):
    i = lax.bitcast_convert_type(v, jnp.int32)
    i = jnp.int32(0x5F3759DF) - lax.shift_right_arithmetic(i, 1)
    y = lax.bitcast_convert_type(i, jnp.float32)
    vh = v * jnp.float32(0.5)
    for _ in range(3):
        y = y * (jnp.float32(1.5) - vh * y * y)
    return y


def _body(ids_hbm, pos_hbm, tt_hbm, wtab, ptab, tttab, lnw, lnb, out,
          widx, pidx, tidx, wbuf, pbuf, ttb, lwb, lbb, sem_w, sem_p, sem_o):
    wid = lax.axis_index("s") * NC + lax.axis_index("c")
    pltpu.sync_copy(tttab, ttb)
    pltpu.sync_copy(lnw, lwb)
    pltpu.sync_copy(lnb, lbb)
    inv_h = jnp.float32(1.0 / H)

    def issue(cc, b):
        bs = wid * TPW + cc * C
        pltpu.sync_copy(ids_hbm.at[pl.ds(bs, C)], widx[b])
        pltpu.sync_copy(pos_hbm.at[pl.ds(bs, C)], pidx[b])
        pltpu.sync_copy(tt_hbm.at[pl.ds(bs, C)], tidx[b])
        pltpu.async_copy(wtab.at[widx[b]], wbuf[b], sem_w[b])
        pltpu.async_copy(ptab.at[pidx[b]], pbuf[b], sem_p[b])

    def wait_gather(b):
        pltpu.make_async_copy(wtab.at[widx[b]], wbuf[b], sem_w[b]).wait()
        pltpu.make_async_copy(ptab.at[pidx[b]], pbuf[b], sem_p[b]).wait()

    def out_slice(cc):
        return out.at[pl.ds(wid * TPW + cc * C, C)]

    def ln(b):
        wb = wbuf[b]
        pb = pbuf[b]
        ttf = tidx[b][pl.ds(0, L)].astype(jnp.float32)
        ttf_s = [ttf[t] for t in range(L)]

        def p1(j, kcarry):
            d0 = j * L
            t0v = ttb[0, pl.ds(d0, L)]
            dtv = ttb[1, pl.ds(d0, L)] - t0v
            new = []
            for t in range(L):
                x = (wb[t, pl.ds(d0, L)] + pb[t, pl.ds(d0, L)]
                     + (dtv * ttf_s[t] + t0v))
                wb[t, pl.ds(d0, L)] = x
                new.append(kcarry[2 * t] + x)
                new.append(kcarry[2 * t + 1] + x * x)
            return tuple(new)

        zero = jnp.zeros((L,), jnp.float32)
        acc = lax.fori_loop(0, HB, p1, (zero,) * (2 * L))
        a_s = []
        c_s = []
        for t in range(L):
            mean = jnp.sum(acc[2 * t]) * inv_h
            var = jnp.sum(acc[2 * t + 1]) * inv_h - mean * mean
            r = _rsqrt_scalar(var + jnp.float32(EPS))
            a_s.append(r)
            c_s.append(-(mean * r))

        def p2(j, kcarry):
            d0 = j * L
            wv = lwb[pl.ds(d0, L)]
            bv = lbb[pl.ds(d0, L)]
            for t in range(L):
                x = wb[t, pl.ds(d0, L)]
                wb[t, pl.ds(d0, L)] = (x * a_s[t] + c_s[t]) * wv + bv
            return kcarry

        lax.fori_loop(0, HB, p2, jnp.int32(0))

    issue(0, 2)
    issue(1, 3)

    def step(k, carry):
        for p in range(NBUF):
            c = NBUF * k + p
            b = (p + 2) % NBUF
            wait_gather(b)
            ln(b)
            pltpu.async_copy(wbuf[b], out_slice(c), sem_o[b])
            b2 = p
            if p < 2:
                @pl.when(k > 0)
                def _():
                    pltpu.make_async_copy(
                        wbuf[b2], out_slice(c - 2), sem_o[b2]).wait()
                issue(c + 2, b2)
            else:
                pltpu.make_async_copy(
                    wbuf[b2], out_slice(c - 2), sem_o[b2]).wait()
                @pl.when(c + 2 < NCH)
                def _():
                    issue(c + 2, b2)
        return carry

    lax.fori_loop(0, NCH // NBUF, step, jnp.int32(0))
    pltpu.make_async_copy(wbuf[0], out_slice(NCH - 2), sem_o[0]).wait()
    pltpu.make_async_copy(wbuf[1], out_slice(NCH - 1), sem_o[1]).wait()


@jax.jit
def kernel(input_ids, seq_lens, position_ids, token_type_ids,
           word_embeddings, position_embeddings, token_type_embeddings,
           ln_weight, ln_bias):
    del seq_lens
    mesh = plsc.VectorSubcoreMesh(core_axis_name="c", subcore_axis_name="s")
    kfn = pl.kernel(
        _body,
        out_type=jax.ShapeDtypeStruct((TOTAL, H), jnp.float32),
        mesh=mesh,
        compiler_params=pltpu.CompilerParams(needs_layout_passes=False),
        scratch_types=[
            [pltpu.VMEM((C,), jnp.int32) for _ in range(NBUF)],
            [pltpu.VMEM((C,), jnp.int32) for _ in range(NBUF)],
            [pltpu.VMEM((C,), jnp.int32) for _ in range(NBUF)],
            [pltpu.VMEM((C, H), jnp.float32) for _ in range(NBUF)],
            [pltpu.VMEM((C, H), jnp.float32) for _ in range(NBUF)],
            pltpu.VMEM((2, H), jnp.float32),
            pltpu.VMEM((H,), jnp.float32),
            pltpu.VMEM((H,), jnp.float32),
            [pltpu.SemaphoreType.DMA for _ in range(NBUF)],
            [pltpu.SemaphoreType.DMA for _ in range(NBUF)],
            [pltpu.SemaphoreType.DMA for _ in range(NBUF)],
        ],
    )
    return kfn(input_ids.astype(jnp.int32), position_ids.astype(jnp.int32),
               token_type_ids.astype(jnp.int32), word_embeddings,
               position_embeddings, token_type_embeddings,
               ln_weight, ln_bias)
